# baseline (speedup 1.0000x reference)
import functools

import jax
import jax.numpy as jnp
from jax import lax
from jax.experimental import pallas as pl
from jax.experimental.pallas import tpu as pltpu
from jax.experimental.pallas import tpu_sc as plsc

VOCAB = 1000000
EMB = 64
BATCH = 16384

NC = 2
NS = 16
NW = NC * NS
B_PER_W = BATCH // NW
ROWS_W = B_PER_W // 2
LANES = 16
VBLK = 512


def _tc_transpose_body(x_ref, o_ref):
    x = x_ref[...]
    y = jnp.transpose(x)
    o_ref[...] = y.reshape(VBLK // 8, 8, EMB)


def _tc_transpose(embT):
    n_blk = (VOCAB + VBLK - 1) // VBLK
    return pl.pallas_call(
        _tc_transpose_body,
        grid=(n_blk,),
        in_specs=[pl.BlockSpec((EMB, VBLK), lambda i: (0, i))],
        out_specs=pl.BlockSpec((VBLK // 8, 8, EMB), lambda i: (i, 0, 0)),
        out_shape=jax.ShapeDtypeStruct((VOCAB // 8, 8, EMB), jnp.float32),
    )(embT)


def _sc_body(emb_t3, emb_c3, tv_hbm, cv_hbm, out_hbm,
             idx_t, idx_c, rows_t, rows_c, sem):
    wid = lax.axis_index("s") * NC + lax.axis_index("c")
    base = wid * B_PER_W

    pltpu.sync_copy(tv_hbm.at[pl.ds(base, B_PER_W)], idx_t)
    pltpu.sync_copy(cv_hbm.at[pl.ds(base, B_PER_W)], idx_c)

    def fetch_body(g, _):
        gb = g * LANES
        vt = idx_t[pl.ds(gb, LANES)]
        vc = idx_c[pl.ds(gb, LANES)]
        for j in range(LANES):
            it = vt[j]
            ic = vc[j]
            i = gb + j
            dst = pl.ds((i & 1) * EMB, EMB)
            pltpu.async_copy(emb_t3.at[it >> 3, it & 7],
                             rows_t.at[i >> 1, dst], sem)
            pltpu.async_copy(emb_c3.at[ic >> 3, ic & 7],
                             rows_c.at[i >> 1, dst], sem)
        return _

    lax.fori_loop(0, B_PER_W // LANES, fetch_body, None)

    dummy = out_hbm.at[pl.ds(wid * ROWS_W, ROWS_W)]
    pltpu.make_async_copy(dummy, rows_t, sem).wait()
    pltpu.make_async_copy(dummy, rows_c, sem).wait()

    def mul_row(r, _):
        for c in range(128 // LANES):
            csl = pl.ds(c * LANES, LANES)
            rows_t[r, csl] = rows_t[r, csl] * rows_c[r, csl]
        return _

    lax.fori_loop(0, ROWS_W, mul_row, None)

    pltpu.sync_copy(rows_t, out_hbm.at[pl.ds(wid * ROWS_W, ROWS_W)])


def _sc_gather_mul(tvec, cvec, emb_t3, emb_c3):
    mesh = plsc.VectorSubcoreMesh(core_axis_name="c", subcore_axis_name="s")
    run = functools.partial(
        pl.kernel,
        mesh=mesh,
        out_type=jax.ShapeDtypeStruct((BATCH // 2, 128), jnp.float32),
        scratch_types=[
            pltpu.VMEM((B_PER_W,), jnp.int32),
            pltpu.VMEM((B_PER_W,), jnp.int32),
            pltpu.VMEM((ROWS_W, 128), jnp.float32),
            pltpu.VMEM((ROWS_W, 128), jnp.float32),
            pltpu.SemaphoreType.DMA,
        ],
    )(_sc_body)
    return run(emb_t3, emb_c3, tvec, cvec)


def _tc_reduce_body(x_ref, o_ref):
    i = pl.program_id(0)
    x = x_ref[...]
    ls = jnp.minimum(x, 0.0) - jnp.log1p(jnp.exp(-jnp.abs(x)))
    s = -jnp.sum(ls) * (1.0 / (BATCH * EMB))

    @pl.when(i == 0)
    def _():
        o_ref[0, 0] = s

    @pl.when(i > 0)
    def _():
        o_ref[0, 0] += s


def _tc_reduce(x):
    n_blk = 8
    rows = x.shape[0] // n_blk
    return pl.pallas_call(
        _tc_reduce_body,
        grid=(n_blk,),
        in_specs=[pl.BlockSpec((rows, 128), lambda i: (i, 0))],
        out_specs=pl.BlockSpec((1, 1), lambda i: (0, 0),
                               memory_space=pltpu.SMEM),
        out_shape=jax.ShapeDtypeStruct((1, 1), jnp.float32),
    )(x)


def kernel(target_vec, context_vec, emb_target, emb_context):
    tvec = target_vec.astype(jnp.int32)
    cvec = context_vec.astype(jnp.int32)
    emb_t3 = emb_target.reshape(VOCAB // 8, 8, EMB)
    emb_c3 = _tc_transpose(emb_context.T)
    prod = _sc_gather_mul(tvec, cvec, emb_t3, emb_c3)
    loss = _tc_reduce(prod)
    return loss.reshape(())


# speedup vs baseline: 2.1665x; 2.1665x over previous
import functools

import jax
import jax.numpy as jnp
from jax import lax
from jax.experimental import pallas as pl
from jax.experimental.pallas import tpu as pltpu
from jax.experimental.pallas import tpu_sc as plsc

VOCAB = 1000000
EMB = 64
BATCH = 16384

NC = 2
NS = 16
NW = NC * NS
B_PER_W = BATCH // NW
ROWS_W = B_PER_W // 2
LANES = 16
VBLK = 2048


def _tc_transpose_body(x_ref, o_ref):
    o_ref[...] = jnp.transpose(x_ref[...])


def _tc_transpose(embT):
    n_blk = (VOCAB + VBLK - 1) // VBLK
    return pl.pallas_call(
        _tc_transpose_body,
        grid=(n_blk,),
        in_specs=[pl.BlockSpec((EMB, VBLK), lambda i: (0, i))],
        out_specs=pl.BlockSpec((VBLK, EMB), lambda i: (i, 0)),
        out_shape=jax.ShapeDtypeStruct((VOCAB, EMB), jnp.float32),
    )(embT)


def _sc_body(emb_t3, emb_c2, tv_hbm, cv_hbm, out_hbm,
             idx_t, idx_c, rows_t, rows_c, sem):
    wid = lax.axis_index("s") * NC + lax.axis_index("c")
    base = wid * B_PER_W

    pltpu.sync_copy(tv_hbm.at[pl.ds(base, B_PER_W)], idx_t)
    pltpu.sync_copy(cv_hbm.at[pl.ds(base, B_PER_W)], idx_c)

    def fetch_body(g, _):
        gb = g * LANES
        vt = idx_t[pl.ds(gb, LANES)]
        vc = idx_c[pl.ds(gb, LANES)]
        for j in range(LANES):
            it = vt[j]
            ic = vc[j]
            i = gb + j
            dst = pl.ds((i & 1) * EMB, EMB)
            pltpu.async_copy(emb_t3.at[it >> 3, it & 7],
                             rows_t.at[i >> 1, dst], sem)
            pltpu.async_copy(emb_c2.at[ic], rows_c.at[i >> 1, dst], sem)
        return _

    lax.fori_loop(0, B_PER_W // LANES, fetch_body, None)

    dummy = out_hbm.at[pl.ds(wid * ROWS_W, ROWS_W)]
    pltpu.make_async_copy(dummy, rows_t, sem).wait()
    pltpu.make_async_copy(dummy, rows_c, sem).wait()

    def mul_row(r, _):
        for c in range(128 // LANES):
            csl = pl.ds(c * LANES, LANES)
            rows_t[r, csl] = rows_t[r, csl] * rows_c[r, csl]
        return _

    lax.fori_loop(0, ROWS_W, mul_row, None)

    pltpu.sync_copy(rows_t, out_hbm.at[pl.ds(wid * ROWS_W, ROWS_W)])


def _sc_gather_mul(tvec, cvec, emb_t3, emb_c2):
    mesh = plsc.VectorSubcoreMesh(core_axis_name="c", subcore_axis_name="s")
    run = functools.partial(
        pl.kernel,
        mesh=mesh,
        out_type=jax.ShapeDtypeStruct((BATCH // 2, 128), jnp.float32),
        scratch_types=[
            pltpu.VMEM((B_PER_W,), jnp.int32),
            pltpu.VMEM((B_PER_W,), jnp.int32),
            pltpu.VMEM((ROWS_W, 128), jnp.float32),
            pltpu.VMEM((ROWS_W, 128), jnp.float32),
            pltpu.SemaphoreType.DMA,
        ],
    )(_sc_body)
    return run(emb_t3, emb_c2, tvec, cvec)


def _tc_reduce_body(x_ref, o_ref):
    i = pl.program_id(0)
    x = x_ref[...]
    ls = jnp.minimum(x, 0.0) - jnp.log1p(jnp.exp(-jnp.abs(x)))
    s = -jnp.sum(ls) * (1.0 / (BATCH * EMB))

    @pl.when(i == 0)
    def _():
        o_ref[0, 0] = s

    @pl.when(i > 0)
    def _():
        o_ref[0, 0] += s


def _tc_reduce(x):
    n_blk = 8
    rows = x.shape[0] // n_blk
    return pl.pallas_call(
        _tc_reduce_body,
        grid=(n_blk,),
        in_specs=[pl.BlockSpec((rows, 128), lambda i: (i, 0))],
        out_specs=pl.BlockSpec((1, 1), lambda i: (0, 0),
                               memory_space=pltpu.SMEM),
        out_shape=jax.ShapeDtypeStruct((1, 1), jnp.float32),
    )(x)


def kernel(target_vec, context_vec, emb_target, emb_context):
    tvec = target_vec.astype(jnp.int32)
    cvec = context_vec.astype(jnp.int32)
    emb_t3 = emb_target.reshape(VOCAB // 8, 8, EMB)
    emb_c2 = _tc_transpose(emb_context.T)
    prod = _sc_gather_mul(tvec, cvec, emb_t3, emb_c2)
    loss = _tc_reduce(prod)
    return loss.reshape(())


# speedup vs baseline: 2.7992x; 1.2920x over previous
import functools

import jax
import jax.numpy as jnp
from jax import lax
from jax.experimental import pallas as pl
from jax.experimental.pallas import tpu as pltpu
from jax.experimental.pallas import tpu_sc as plsc

VOCAB = 1000000
EMB = 64
BATCH = 16384

NC = 2
NS = 16
NW = NC * NS
B_PER_W = BATCH // NW
ROWS_W = B_PER_W // 2
LANES = 16
VBLK = 8192


def _tc_transpose_body(x_ref, o_ref):
    o_ref[...] = jnp.transpose(x_ref[...])


def _tc_transpose(embT):
    n_blk = (VOCAB + VBLK - 1) // VBLK
    return pl.pallas_call(
        _tc_transpose_body,
        grid=(n_blk,),
        in_specs=[pl.BlockSpec((EMB, VBLK), lambda i: (0, i))],
        out_specs=pl.BlockSpec((VBLK, EMB), lambda i: (i, 0)),
        out_shape=jax.ShapeDtypeStruct((VOCAB, EMB), jnp.float32),
    )(embT)


def _sc_body(emb_t3, emb_c2, tv_hbm, cv_hbm, out_hbm,
             idx_t, idx_c, rows_t, rows_c, sem):
    wid = lax.axis_index("s") * NC + lax.axis_index("c")
    base = wid * B_PER_W

    pltpu.sync_copy(tv_hbm.at[pl.ds(base, B_PER_W)], idx_t)
    pltpu.sync_copy(cv_hbm.at[pl.ds(base, B_PER_W)], idx_c)

    def fetch_body(g, _):
        gb = g * LANES
        vt = idx_t[pl.ds(gb, LANES)]
        vc = idx_c[pl.ds(gb, LANES)]
        for j in range(LANES):
            it = vt[j]
            ic = vc[j]
            i = gb + j
            dst = pl.ds((i & 1) * EMB, EMB)
            pltpu.async_copy(emb_t3.at[it >> 3, it & 7],
                             rows_t.at[i >> 1, dst], sem)
            pltpu.async_copy(emb_c2.at[ic], rows_c.at[i >> 1, dst], sem)
        return _

    lax.fori_loop(0, B_PER_W // LANES, fetch_body, None)

    dummy = out_hbm.at[pl.ds(wid * ROWS_W, ROWS_W)]
    pltpu.make_async_copy(dummy, rows_t, sem).wait()
    pltpu.make_async_copy(dummy, rows_c, sem).wait()

    def mul_row(r, _):
        for c in range(128 // LANES):
            csl = pl.ds(c * LANES, LANES)
            rows_t[r, csl] = rows_t[r, csl] * rows_c[r, csl]
        return _

    lax.fori_loop(0, ROWS_W, mul_row, None)

    pltpu.sync_copy(rows_t, out_hbm.at[pl.ds(wid * ROWS_W, ROWS_W)])


def _sc_gather_mul(tvec, cvec, emb_t3, emb_c2):
    mesh = plsc.VectorSubcoreMesh(core_axis_name="c", subcore_axis_name="s")
    run = functools.partial(
        pl.kernel,
        mesh=mesh,
        out_type=jax.ShapeDtypeStruct((BATCH // 2, 128), jnp.float32),
        scratch_types=[
            pltpu.VMEM((B_PER_W,), jnp.int32),
            pltpu.VMEM((B_PER_W,), jnp.int32),
            pltpu.VMEM((ROWS_W, 128), jnp.float32),
            pltpu.VMEM((ROWS_W, 128), jnp.float32),
            pltpu.SemaphoreType.DMA,
        ],
    )(_sc_body)
    return run(emb_t3, emb_c2, tvec, cvec)


def _tc_reduce_body(x_ref, o_ref):
    i = pl.program_id(0)
    x = x_ref[...]
    ls = jnp.minimum(x, 0.0) - jnp.log1p(jnp.exp(-jnp.abs(x)))
    s = -jnp.sum(ls) * (1.0 / (BATCH * EMB))

    @pl.when(i == 0)
    def _():
        o_ref[0, 0] = s

    @pl.when(i > 0)
    def _():
        o_ref[0, 0] += s


def _tc_reduce(x):
    n_blk = 8
    rows = x.shape[0] // n_blk
    return pl.pallas_call(
        _tc_reduce_body,
        grid=(n_blk,),
        in_specs=[pl.BlockSpec((rows, 128), lambda i: (i, 0))],
        out_specs=pl.BlockSpec((1, 1), lambda i: (0, 0),
                               memory_space=pltpu.SMEM),
        out_shape=jax.ShapeDtypeStruct((1, 1), jnp.float32),
    )(x)


def kernel(target_vec, context_vec, emb_target, emb_context):
    tvec = target_vec.astype(jnp.int32)
    cvec = context_vec.astype(jnp.int32)
    emb_t3 = emb_target.reshape(VOCAB // 8, 8, EMB)
    emb_c2 = _tc_transpose(emb_context.T)
    prod = _sc_gather_mul(tvec, cvec, emb_t3, emb_c2)
    loss = _tc_reduce(prod)
    return loss.reshape(())


# speedup vs baseline: 3.4062x; 1.2168x over previous
import functools

import jax
import jax.numpy as jnp
from jax import lax
from jax.experimental import pallas as pl
from jax.experimental.pallas import tpu as pltpu
from jax.experimental.pallas import tpu_sc as plsc

VOCAB = 1000000
EMB = 64
BATCH = 16384

NC = 2
NS = 16
NW = NC * NS
B_PER_W = BATCH // NW
ROWS_W = B_PER_W // 2
LANES = 16
VBLK = 8192


def _tc_transpose_body(x_ref, o_ref):
    y = jnp.transpose(x_ref[...])
    o_ref[...] = jnp.concatenate(
        [y[:VBLK // 2], y[VBLK // 2:]], axis=1)


def _tc_transpose(embT):
    n_blk = (VOCAB + VBLK - 1) // VBLK
    return pl.pallas_call(
        _tc_transpose_body,
        grid=(n_blk,),
        in_specs=[pl.BlockSpec((EMB, VBLK), lambda i: (0, i))],
        out_specs=pl.BlockSpec((VBLK // 2, 2 * EMB), lambda i: (i, 0)),
        out_shape=jax.ShapeDtypeStruct((n_blk * (VBLK // 2), 2 * EMB),
                                       jnp.float32),
    )(embT)


def _sc_body(emb_t3, emb_c2, tv_hbm, cv_hbm, out_hbm,
             idx_t, idx_c, rows_t, rows_c, sem):
    wid = lax.axis_index("s") * NC + lax.axis_index("c")
    base = wid * B_PER_W

    pltpu.sync_copy(tv_hbm.at[pl.ds(base, B_PER_W)], idx_t)
    pltpu.sync_copy(cv_hbm.at[pl.ds(base, B_PER_W)], idx_c)

    def fetch_body(g, _):
        gb = g * LANES
        vt = idx_t[pl.ds(gb, LANES)]
        vc = idx_c[pl.ds(gb, LANES)]
        for j in range(LANES):
            it = vt[j]
            ic = vc[j]
            i = gb + j
            dst = pl.ds((i & 1) * EMB, EMB)
            pltpu.async_copy(emb_t3.at[it >> 3, it & 7],
                             rows_t.at[i >> 1, dst], sem)
            rc = ((ic >> 13) << 12) + (ic & 4095)
            pltpu.async_copy(emb_c2.at[rc], rows_c.at[i], sem)
        return _

    lax.fori_loop(0, B_PER_W // LANES, fetch_body, None)

    pltpu.make_async_copy(out_hbm.at[pl.ds(wid * ROWS_W, ROWS_W)],
                          rows_t, sem).wait()
    pltpu.make_async_copy(out_hbm.at[pl.ds(0, B_PER_W)],
                          rows_c, sem).wait()

    def mul_grp(g, _):
        gb = g * LANES
        vc = idx_c[pl.ds(gb, LANES)]
        for j in range(LANES):
            i = gb + j
            hc = ((vc[j] >> 12) & 1) * EMB
            for c in range(EMB // LANES):
                tsl = pl.ds((i & 1) * EMB + c * LANES, LANES)
                csl = pl.ds(hc + c * LANES, LANES)
                rows_t[i >> 1, tsl] = rows_t[i >> 1, tsl] * rows_c[i, csl]
        return _

    lax.fori_loop(0, B_PER_W // LANES, mul_grp, None)

    pltpu.sync_copy(rows_t, out_hbm.at[pl.ds(wid * ROWS_W, ROWS_W)])


def _sc_gather_mul(tvec, cvec, emb_t3, emb_c2):
    mesh = plsc.VectorSubcoreMesh(core_axis_name="c", subcore_axis_name="s")
    run = functools.partial(
        pl.kernel,
        mesh=mesh,
        out_type=jax.ShapeDtypeStruct((BATCH // 2, 128), jnp.float32),
        scratch_types=[
            pltpu.VMEM((B_PER_W,), jnp.int32),
            pltpu.VMEM((B_PER_W,), jnp.int32),
            pltpu.VMEM((ROWS_W, 128), jnp.float32),
            pltpu.VMEM((B_PER_W, 128), jnp.float32),
            pltpu.SemaphoreType.DMA,
        ],
    )(_sc_body)
    return run(emb_t3, emb_c2, tvec, cvec)


def _tc_reduce_body(x_ref, o_ref):
    i = pl.program_id(0)
    x = x_ref[...]
    ls = jnp.minimum(x, 0.0) - jnp.log1p(jnp.exp(-jnp.abs(x)))
    s = -jnp.sum(ls) * (1.0 / (BATCH * EMB))

    @pl.when(i == 0)
    def _():
        o_ref[0, 0] = s

    @pl.when(i > 0)
    def _():
        o_ref[0, 0] += s


def _tc_reduce(x):
    n_blk = 8
    rows = x.shape[0] // n_blk
    return pl.pallas_call(
        _tc_reduce_body,
        grid=(n_blk,),
        in_specs=[pl.BlockSpec((rows, 128), lambda i: (i, 0))],
        out_specs=pl.BlockSpec((1, 1), lambda i: (0, 0),
                               memory_space=pltpu.SMEM),
        out_shape=jax.ShapeDtypeStruct((1, 1), jnp.float32),
    )(x)


def kernel(target_vec, context_vec, emb_target, emb_context):
    tvec = target_vec.astype(jnp.int32)
    cvec = context_vec.astype(jnp.int32)
    emb_t3 = emb_target.reshape(VOCAB // 8, 8, EMB)
    emb_c2 = _tc_transpose(emb_context.T)
    prod = _sc_gather_mul(tvec, cvec, emb_t3, emb_c2)
    loss = _tc_reduce(prod)
    return loss.reshape(())
